# baseline (speedup 1.0000x reference)
import functools

import jax
import jax.numpy as jnp
import numpy as np
from jax import lax
from jax.experimental import pallas as pl
from jax.experimental.pallas import tpu as pltpu
from jax.experimental.pallas import tpu_sc as plsc

KNN = 32
N = 4096
B = 2
NW = 32
RPW = B * N // NW
CAP = 1024
INF = np.float32(np.inf)
NEG_INF = np.float32(-np.inf)
POS_BIG = np.int32(1 << 30)


def _bf16_round(v):
    bits = lax.bitcast_convert_type(v, jnp.int32)
    r = (bits + 0x7FFF + ((bits >> 16) & 1)) & np.int32(-65536)
    return lax.bitcast_convert_type(r, jnp.float32)


def _sc_body(xt_hbm, c_hbm, idx_hbm, msk_hbm,
             xraw_v, xgb_v, sqm_v, c_v, d2_v,
             cval_v, cidx_v, mf_v, oidx_v, omsk_v):
    cid = lax.axis_index("c")
    sid = lax.axis_index("s")
    wid = sid * 2 + cid
    b = wid // 16
    row0 = (wid % 16) * RPW
    lane = lax.iota(jnp.int32, 16)

    pltpu.sync_copy(xt_hbm.at[b], xraw_v)
    pltpu.sync_copy(c_hbm.at[b], c_v)

    def col_body(t, carry):
        for u in range(4):
            o = t * 64 + u * 16
            xg = []
            for d in range(3):
                s = ((xraw_v[d, pl.ds(o, 16)] + xraw_v[3 + d, pl.ds(o, 16)])
                     + xraw_v[6 + d, pl.ds(o, 16)]
                     + xraw_v[9 + d, pl.ds(o, 16)]) * 0.25
                xg.append(s)
            sq = (xg[0] * xg[0] + xg[1] * xg[1]) + xg[2] * xg[2]
            cc = c_v[pl.ds(o, 16)]
            sqm_v[pl.ds(o, 16)] = jnp.where(cc > 0, sq, INF)
            for d in range(3):
                xgb_v[d, pl.ds(o, 16)] = _bf16_round(xg[d])
        return carry
    lax.fori_loop(0, N // 64, col_body, 0)

    def mf_body(t, p):
        o = t * 16
        mm = c_v[pl.ds(o, 16)] <= 0
        cnt = jnp.max(plsc.all_reduce_population_count(mm))
        plsc.store_compressed(mf_v.at[pl.ds(p, 16)], lane + o, mask=mm)
        return jnp.minimum(p + cnt, np.int32(32))
    lax.fori_loop(0, N // 16, mf_body, np.int32(0))

    def row_body(i, carry):
        ri = row0 + i
        co = (ri // 16) * 16
        onehot = lane == (ri % 16)
        ci = jnp.max(jnp.where(onehot, c_v[pl.ds(co, 16)],
                               np.int32(-2147483647)))

        @pl.when(ci <= 0)
        def _():
            oidx_v[i, pl.ds(0, 16)] = lane
            oidx_v[i, pl.ds(16, 16)] = lane + 16
            omsk_v[i, pl.ds(0, 16)] = jnp.zeros((16,), jnp.float32)
            omsk_v[i, pl.ds(16, 16)] = jnp.zeros((16,), jnp.float32)

        @pl.when(ci > 0)
        def _():
            xi = []
            for d in range(3):
                ch = xgb_v[d, pl.ds(co, 16)]
                xi.append(jnp.sum(jnp.where(onehot, ch, np.float32(0.0))))
            sqi = jnp.sum(jnp.where(onehot, sqm_v[pl.ds(co, 16)],
                                    np.float32(0.0)))

            def grp_body(g, t0):
                gm = jnp.full((16,), INF, jnp.float32)
                for u in range(8):
                    o = g * 128 + u * 16
                    v = (xi[0] * xgb_v[0, pl.ds(o, 16)]
                         + xi[1] * xgb_v[1, pl.ds(o, 16)]) \
                        + xi[2] * xgb_v[2, pl.ds(o, 16)]
                    d2 = jnp.maximum((sqi + sqm_v[pl.ds(o, 16)]) - 2.0 * v,
                                     0.0)
                    d2_v[pl.ds(o, 16)] = d2
                    gm = jnp.minimum(gm, d2)
                return jnp.maximum(t0, jnp.min(gm))
            t0 = lax.fori_loop(0, 32, grp_body, NEG_INF)

            def collect(t, p):
                for u in range(4):
                    o = t * 64 + u * 16
                    d2c = d2_v[pl.ds(o, 16)]
                    mm = (d2c <= t0) & (d2c < INF)
                    cnt = jnp.max(plsc.all_reduce_population_count(mm))
                    plsc.store_compressed(cval_v.at[pl.ds(p, 16)], d2c,
                                          mask=mm)
                    plsc.store_compressed(cidx_v.at[pl.ds(p, 16)],
                                          lane + o, mask=mm)
                    p = jnp.minimum(p + cnt, np.int32(CAP - 16))
                return p
            nv = lax.fori_loop(0, N // 64, collect, np.int32(0))
            cval_v[pl.ds(nv, 16)] = jnp.full((16,), INF, jnp.float32)
            nb = (nv + 15) // 16

            def sel_k(k, acc):
                oi0, oi1, om0, om1 = acc

                def scan_q(q, mp):
                    mn, pv = mp
                    v = cval_v[pl.ds(q * 16, 16)]
                    lt = v < mn
                    return (jnp.where(lt, v, mn),
                            jnp.where(lt, lane + q * 16, pv))
                mn, pv = lax.fori_loop(
                    0, nb, scan_q,
                    (jnp.full((16,), INF, jnp.float32),
                     jnp.full((16,), POS_BIG, jnp.int32)))
                m = jnp.min(mn)
                pos = jnp.min(jnp.where(mn == m, pv, POS_BIG))
                is_real = m < INF
                pos_s = jnp.where(is_real, pos, 0)
                posv = jnp.zeros((16,), jnp.int32) + pos_s
                jv = plsc.load_gather(cidx_v, [posv])
                fv = plsc.load_gather(
                    mf_v, [jnp.zeros((16,), jnp.int32)
                           + jnp.where(is_real, 0, k - nv)])
                outj = jnp.where(is_real, jv, fv)
                plsc.store_scatter(cval_v, [posv],
                                   jnp.full((16,), INF, jnp.float32),
                                   mask=(lane == 0) & is_real)
                sel = lane == (k % 16)
                first = k < 16
                mv = jnp.where(is_real, np.float32(1.0), np.float32(0.0))
                mvv = jnp.zeros((16,), jnp.float32) + mv
                oi0 = jnp.where(sel & first, outj, oi0)
                oi1 = jnp.where(sel & (~first), outj, oi1)
                om0 = jnp.where(sel & first, mvv, om0)
                om1 = jnp.where(sel & (~first), mvv, om1)
                return (oi0, oi1, om0, om1)

            z_i = jnp.zeros((16,), jnp.int32)
            z_f = jnp.zeros((16,), jnp.float32)
            oi0, oi1, om0, om1 = lax.fori_loop(0, KNN, sel_k,
                                               (z_i, z_i, z_f, z_f))
            oidx_v[i, pl.ds(0, 16)] = oi0
            oidx_v[i, pl.ds(16, 16)] = oi1
            omsk_v[i, pl.ds(0, 16)] = om0
            omsk_v[i, pl.ds(16, 16)] = om1
        return carry
    lax.fori_loop(0, RPW, row_body, 0)

    pltpu.sync_copy(oidx_v, idx_hbm.at[pl.ds(wid * RPW, RPW)])
    pltpu.sync_copy(omsk_v, msk_hbm.at[pl.ds(wid * RPW, RPW)])


@jax.jit
def kernel(X, C):
    Xt = X.reshape(B, N, 12).transpose(0, 2, 1)
    Ci = C.astype(jnp.int32)
    mesh = plsc.VectorSubcoreMesh(core_axis_name="c", subcore_axis_name="s")
    run = functools.partial(
        pl.kernel,
        mesh=mesh,
        compiler_params=pltpu.CompilerParams(needs_layout_passes=False,
                                             use_tc_tiling_on_sc=False),
        out_type=[jax.ShapeDtypeStruct((B * N, KNN), jnp.int32),
                  jax.ShapeDtypeStruct((B * N, KNN), jnp.float32)],
        scratch_types=[
            pltpu.VMEM((12, N), jnp.float32),
            pltpu.VMEM((3, N), jnp.float32),
            pltpu.VMEM((N,), jnp.float32),
            pltpu.VMEM((N,), jnp.int32),
            pltpu.VMEM((N,), jnp.float32),
            pltpu.VMEM((CAP,), jnp.float32),
            pltpu.VMEM((CAP,), jnp.int32),
            pltpu.VMEM((48,), jnp.int32),
            pltpu.VMEM((RPW, KNN), jnp.int32),
            pltpu.VMEM((RPW, KNN), jnp.float32),
        ],
    )(_sc_body)
    idx_flat, msk_flat = run(Xt, Ci)
    return idx_flat.reshape(B, N, KNN), msk_flat.reshape(B, N, KNN)
